# Initial kernel scaffold; baseline (speedup 1.0000x reference)
import jax
import jax.numpy as jnp
from jax.experimental import pallas as pl


def kernel(latents, embedding_weight):
    raise NotImplementedError("write your pallas kernel here")


# speedup vs baseline: 3.7998x; 3.7998x over previous
import functools

import jax
import jax.numpy as jnp
from jax import lax
from jax.experimental import pallas as pl
from jax.experimental.pallas import tpu as pltpu
from jax.experimental.pallas import tpu_sc as plsc

K = 8192
D = 64
TM = 256
KC = 2048
LANES = 128

NUM_WORKERS = 32
B_TOTAL = 16 * 576
B_PER_W = B_TOTAL // NUM_WORKERS
GCHUNK = 96


def _argmin_body(x_ref, zn_ref, e_ref, en_ref, out_ref):
    x = x_ref[...]
    zn = zn_ref[...]

    best_v = None
    best_i = None
    for c in range(K // KC):
        e_chunk = e_ref[pl.ds(c * KC, KC), :]
        en_chunk = en_ref[:, pl.ds(c * KC, KC)]
        mm = lax.dot_general(x, e_chunk,
                             dimension_numbers=(((1,), (1,)), ((), ())),
                             preferred_element_type=jnp.float32)
        dist = (zn + en_chunk) - 2.0 * mm
        for s in range(KC // LANES):
            cand = dist[:, s * LANES:(s + 1) * LANES]
            cidx = (c * KC + s * LANES
                    + lax.broadcasted_iota(jnp.int32, (TM, LANES), 1))
            if best_v is None:
                best_v, best_i = cand, cidx
            else:
                m = cand < best_v
                best_v = jnp.where(m, cand, best_v)
                best_i = jnp.where(m, cidx, best_i)

    gm = jnp.min(best_v, axis=1, keepdims=True)
    tie = best_v == gm
    masked_i = jnp.where(tie, best_i, jnp.int32(2 ** 30))
    out_ref[...] = jnp.min(masked_i, axis=1, keepdims=True)


def _compute_indices(flat, zn, e, en):
    grid = flat.shape[0] // TM
    return pl.pallas_call(
        _argmin_body,
        grid=(grid,),
        in_specs=[
            pl.BlockSpec((TM, D), lambda i: (i, 0)),
            pl.BlockSpec((TM, 1), lambda i: (i, 0)),
            pl.BlockSpec((K, D), lambda i: (0, 0)),
            pl.BlockSpec((1, K), lambda i: (0, 0)),
        ],
        out_specs=pl.BlockSpec((TM, 1), lambda i: (i, 0)),
        out_shape=jax.ShapeDtypeStruct((flat.shape[0], 1), jnp.int32),
    )(flat, zn, e, en)


DPAD = 128


@functools.cache
def _make_sc_gather():
    mesh = plsc.VectorSubcoreMesh(core_axis_name="c", subcore_axis_name="s")

    @functools.partial(
        pl.kernel,
        mesh=mesh,
        out_type=jax.ShapeDtypeStruct((B_TOTAL, DPAD), jnp.float32),
        scratch_types=[
            pltpu.VMEM((B_PER_W,), jnp.int32),
            pltpu.VMEM((B_PER_W, DPAD), jnp.float32),
            pltpu.SemaphoreType.DMA,
        ],
    )
    def _sc_gather(table_hbm, idx_hbm, out_hbm, idx_v, rows_v, sem):
        wid = lax.axis_index("s") * 2 + lax.axis_index("c")
        base = wid * B_PER_W
        pltpu.sync_copy(idx_hbm.at[pl.ds(base, B_PER_W)], idx_v)
        copies = []
        for ch in range(B_PER_W // GCHUNK):
            copies.append(pltpu.async_copy(
                table_hbm.at[idx_v.at[pl.ds(ch * GCHUNK, GCHUNK)]],
                rows_v.at[pl.ds(ch * GCHUNK, GCHUNK)],
                sem))
        for cp in copies:
            cp.wait()
        pltpu.sync_copy(rows_v, out_hbm.at[pl.ds(base, B_PER_W)])

    return _sc_gather


def kernel(latents, embedding_weight):
    d = embedding_weight.shape[1]
    latents_shape = latents.shape
    flat = latents.reshape(-1, d)
    zn = jnp.sum(flat ** 2, axis=1, keepdims=True)
    en = jnp.sum(embedding_weight ** 2, axis=1)

    idx2d = _compute_indices(flat, zn, embedding_weight, en.reshape(1, K))
    idx = idx2d.reshape(-1)

    table_pad = jnp.concatenate(
        [embedding_weight, jnp.zeros_like(embedding_weight)], axis=1)
    quant_pad = _make_sc_gather()(table_pad, idx)
    quantized = quant_pad[:, :D].reshape(latents_shape)
    return latents + lax.stop_gradient(quantized - latents)


# speedup vs baseline: 4.4075x; 1.1599x over previous
import functools

import jax
import jax.numpy as jnp
from jax import lax
from jax.experimental import pallas as pl
from jax.experimental.pallas import tpu as pltpu
from jax.experimental.pallas import tpu_sc as plsc

K = 8192
D = 64
TM = 512
KC = 2048
LANES = 128

NUM_WORKERS = 32
B_TOTAL = 16 * 576
B_PER_W = B_TOTAL // NUM_WORKERS
GCHUNK = 96


def _argmin_body(x_ref, zn_ref, e_ref, en_ref, out_ref):
    xm2 = -2.0 * x_ref[...]
    zn = zn_ref[...]

    best_v = None
    best_i = None
    for c in range(K // KC):
        e_chunk = e_ref[pl.ds(c * KC, KC), :]
        en_chunk = en_ref[:, pl.ds(c * KC, KC)]
        mm2 = lax.dot_general(xm2, e_chunk,
                              dimension_numbers=(((1,), (1,)), ((), ())),
                              preferred_element_type=jnp.float32)
        dist = (zn + en_chunk) + mm2
        for s in range(KC // LANES):
            cand = dist[:, s * LANES:(s + 1) * LANES]
            cidx = (c * KC + s * LANES
                    + lax.broadcasted_iota(jnp.int32, (TM, LANES), 1))
            if best_v is None:
                best_v, best_i = cand, cidx
            else:
                m = cand < best_v
                best_v = jnp.where(m, cand, best_v)
                best_i = jnp.where(m, cidx, best_i)

    gm = jnp.min(best_v, axis=1, keepdims=True)
    tie = best_v == gm
    masked_i = jnp.where(tie, best_i, jnp.int32(2 ** 30))
    out_ref[...] = jnp.min(masked_i, axis=1, keepdims=True)


def _compute_indices(flat, zn, e, en):
    grid = flat.shape[0] // TM
    return pl.pallas_call(
        _argmin_body,
        grid=(grid,),
        in_specs=[
            pl.BlockSpec((TM, D), lambda i: (i, 0)),
            pl.BlockSpec((TM, 1), lambda i: (i, 0)),
            pl.BlockSpec((K, D), lambda i: (0, 0)),
            pl.BlockSpec((1, K), lambda i: (0, 0)),
        ],
        out_specs=pl.BlockSpec((TM, 1), lambda i: (i, 0)),
        out_shape=jax.ShapeDtypeStruct((flat.shape[0], 1), jnp.int32),
    )(flat, zn, e, en)


DPAD = 128


@functools.cache
def _make_sc_gather():
    mesh = plsc.VectorSubcoreMesh(core_axis_name="c", subcore_axis_name="s")

    @functools.partial(
        pl.kernel,
        mesh=mesh,
        out_type=jax.ShapeDtypeStruct((B_TOTAL, DPAD), jnp.float32),
        scratch_types=[
            pltpu.VMEM((B_PER_W,), jnp.int32),
            pltpu.VMEM((B_PER_W, DPAD), jnp.float32),
            pltpu.SemaphoreType.DMA,
        ],
    )
    def _sc_gather(table_hbm, idx_hbm, out_hbm, idx_v, rows_v, sem):
        wid = lax.axis_index("s") * 2 + lax.axis_index("c")
        base = wid * B_PER_W
        pltpu.sync_copy(idx_hbm.at[pl.ds(base, B_PER_W)], idx_v)
        copies = []
        for ch in range(B_PER_W // GCHUNK):
            copies.append(pltpu.async_copy(
                table_hbm.at[idx_v.at[pl.ds(ch * GCHUNK, GCHUNK)]],
                rows_v.at[pl.ds(ch * GCHUNK, GCHUNK)],
                sem))
        for cp in copies:
            cp.wait()
        pltpu.sync_copy(rows_v, out_hbm.at[pl.ds(base, B_PER_W)])

    return _sc_gather


def kernel(latents, embedding_weight):
    d = embedding_weight.shape[1]
    latents_shape = latents.shape
    flat = latents.reshape(-1, d)
    zn = jnp.sum(flat ** 2, axis=1, keepdims=True)
    en = jnp.sum(embedding_weight ** 2, axis=1)

    idx2d = _compute_indices(flat, zn, embedding_weight, en.reshape(1, K))
    idx = idx2d.reshape(-1)

    table_pad = jnp.concatenate(
        [embedding_weight, jnp.zeros_like(embedding_weight)], axis=1)
    quant_pad = _make_sc_gather()(table_pad, idx)
    quantized = quant_pad[:, :D].reshape(latents_shape)
    return latents + lax.stop_gradient(quantized - latents)
